# Initial kernel scaffold; baseline (speedup 1.0000x reference)
import jax
import jax.numpy as jnp
from jax.experimental import pallas as pl


def kernel(x, edge_index, batch, edge_weight, W_enc, b_enc, W_cls, b_cls):
    raise NotImplementedError("write your pallas kernel here")


# speedup vs baseline: 4.0001x; 4.0001x over previous
import functools

import jax
import jax.numpy as jnp
from jax import lax
from jax.experimental import pallas as pl
from jax.experimental.pallas import tpu as pltpu
from jax.experimental.pallas import tpu_sc as plsc

N = 10000
E = 320000
D = 128
C = 10
G = 64

NC = 2
NS = 16
NW = NC * NS
EPW = E // NW
K = 80
CHUNKS = EPW // K
RPS = 624
RTAIL = N - RPS * NS
LANES = 16


def _sc_body(x_hbm, src_hbm, dst_hbm, w_hbm, zero_hbm, out_hbm,
             src_v, dst_v, w_v, rows_v, agg_sh, sem):
    c = lax.axis_index("c")
    s = lax.axis_index("s")
    wid = c * NS + s

    pltpu.sync_copy(zero_hbm.at[pl.ds(s * RPS, RPS)],
                    agg_sh.at[pl.ds(s * RPS, RPS)])

    @pl.when(s == NS - 1)
    def _zero_tail():
        pltpu.sync_copy(zero_hbm.at[pl.ds(RPS * NS, RTAIL)],
                        agg_sh.at[pl.ds(RPS * NS, RTAIL)])

    plsc.subcore_barrier()

    base0 = wid * EPW

    def chunk_body(g, carry):
        base = base0 + g * K
        pltpu.sync_copy(src_hbm.at[pl.ds(base, K)], src_v)
        pltpu.sync_copy(dst_hbm.at[pl.ds(base, K)], dst_v)
        pltpu.sync_copy(w_hbm.at[pl.ds(base, K)], w_v.at[pl.ds(8, K)])
        pltpu.async_copy(x_hbm.at[src_v], rows_v, sem).wait()
        for e in range(K):
            eidx = jnp.full((LANES,), e + 8, dtype=jnp.int32)
            wspl = plsc.load_gather(w_v, [eidx])
            for j in range(D // LANES):
                sl = rows_v[e, pl.ds(j * LANES, LANES)]
                rows_v[e, pl.ds(j * LANES, LANES)] = sl * wspl
        pltpu.sync_copy(rows_v, agg_sh.at[dst_v], add=True)
        return carry

    lax.fori_loop(0, CHUNKS, chunk_body, 0)
    plsc.subcore_barrier()

    pltpu.sync_copy(agg_sh.at[pl.ds(s * RPS, RPS)],
                    out_hbm.at[c, pl.ds(s * RPS, RPS)])

    @pl.when(s == NS - 1)
    def _flush_tail():
        pltpu.sync_copy(agg_sh.at[pl.ds(RPS * NS, RTAIL)],
                        out_hbm.at[c, pl.ds(RPS * NS, RTAIL)])


_sc_aggregate = pl.kernel(
    _sc_body,
    out_type=jax.ShapeDtypeStruct((NC, N, D), jnp.float32),
    mesh=plsc.VectorSubcoreMesh(
        core_axis_name="c", subcore_axis_name="s",
        num_cores=NC, num_subcores=NS),
    scratch_types=[
        pltpu.VMEM((K,), jnp.int32),
        pltpu.VMEM((K,), jnp.int32),
        pltpu.VMEM((K + 8,), jnp.float32),
        pltpu.VMEM((K, D), jnp.float32),
        pltpu.VMEM_SHARED((N, D), jnp.float32),
        pltpu.SemaphoreType.DMA,
    ],
    compiler_params=pltpu.CompilerParams(needs_layout_passes=False),
)


def _tc_head_body(agg_ref, batch_ref, wenc_ref, benc_ref, wcls_ref, bcls_ref,
                  out_ref):
    agg = agg_ref[0] + agg_ref[1]
    h = jnp.dot(agg, wenc_ref[...], preferred_element_type=jnp.float32)
    h = jnp.maximum(h + benc_ref[...], 0.0)
    bt = batch_ref[...]
    gids = lax.broadcasted_iota(jnp.int32, (G, N), 0)
    oh = (gids == bt).astype(jnp.float32)
    pooled_sum = jnp.dot(oh, h, preferred_element_type=jnp.float32)
    counts = jnp.sum(oh, axis=1, keepdims=True)
    pooled = pooled_sum / jnp.maximum(counts, 1.0)
    logits = jnp.dot(pooled, wcls_ref[...],
                     preferred_element_type=jnp.float32) + bcls_ref[...]
    out_ref[...] = logits


_tc_head = pl.pallas_call(
    _tc_head_body,
    out_shape=jax.ShapeDtypeStruct((G, C), jnp.float32),
)


def kernel(x, edge_index, batch, edge_weight, W_enc, b_enc, W_cls, b_cls):
    src = edge_index[0]
    dst = edge_index[1]
    zeros_nd = jnp.zeros((N, D), jnp.float32)
    agg2 = _sc_aggregate(x, src, dst, edge_weight, zeros_nd)
    return _tc_head(agg2, batch.reshape(1, N), W_enc, b_enc.reshape(1, D),
                    W_cls, b_cls.reshape(1, C))


# speedup vs baseline: 4.9266x; 1.2316x over previous
import functools

import jax
import jax.numpy as jnp
from jax import lax
from jax.experimental import pallas as pl
from jax.experimental.pallas import tpu as pltpu
from jax.experimental.pallas import tpu_sc as plsc

N = 10000
E = 320000
D = 128
C = 10
G = 64

NC = 2
NS = 16
NW = NC * NS
EPW = E // NW
K = 80
CHUNKS = EPW // K
RPS = 624
RTAIL = N - RPS * NS
LANES = 16


def _sc_body(x_hbm, sd_hbm, w_hbm, zero_hbm, out_hbm,
             sd0_v, sd1_v, w_v, rows0_v, rows1_v, agg_sh, sem0, sem1):
    c = lax.axis_index("c")
    s = lax.axis_index("s")
    wid = c * NS + s

    pltpu.sync_copy(w_hbm.at[pl.ds(wid * EPW, EPW)], w_v.at[pl.ds(8, EPW)])

    pltpu.sync_copy(zero_hbm.at[pl.ds(s * RPS, RPS)],
                    agg_sh.at[pl.ds(s * RPS, RPS)])

    @pl.when(s == NS - 1)
    def _zero_tail():
        pltpu.sync_copy(zero_hbm.at[pl.ds(RPS * NS, RTAIL)],
                        agg_sh.at[pl.ds(RPS * NS, RTAIL)])

    plsc.subcore_barrier()

    def load_sd(g, sd_v):
        pltpu.sync_copy(sd_hbm.at[wid, g], sd_v)

    def gather(sd_v, rows_v, sem):
        return pltpu.make_async_copy(x_hbm.at[sd_v.at[0]], rows_v, sem)

    def scale_and_scatter(g, rows_v, sd_v):
        wbase = g * K + 8
        for e in range(K):
            eidx = jnp.full((LANES,), e, dtype=jnp.int32) + wbase
            wspl = plsc.load_gather(w_v, [eidx])
            for j in range(D // LANES):
                sl = rows_v[e, pl.ds(j * LANES, LANES)]
                rows_v[e, pl.ds(j * LANES, LANES)] = sl * wspl
        pltpu.sync_copy(rows_v, agg_sh.at[sd_v.at[1]], add=True)

    load_sd(0, sd0_v)
    gather(sd0_v, rows0_v, sem0).start()

    def pair_body(h, carry):
        g0 = h * 2
        load_sd(g0 + 1, sd1_v)
        gather(sd1_v, rows1_v, sem1).start()
        gather(sd0_v, rows0_v, sem0).wait()
        scale_and_scatter(g0, rows0_v, sd0_v)
        load_sd(g0 + 2, sd0_v)
        gather(sd0_v, rows0_v, sem0).start()
        gather(sd1_v, rows1_v, sem1).wait()
        scale_and_scatter(g0 + 1, rows1_v, sd1_v)
        return carry

    lax.fori_loop(0, CHUNKS // 2, pair_body, 0)
    gather(sd0_v, rows0_v, sem0).wait()
    scale_and_scatter(CHUNKS - 1, rows0_v, sd0_v)
    plsc.subcore_barrier()

    pltpu.sync_copy(agg_sh.at[pl.ds(s * RPS, RPS)],
                    out_hbm.at[c, pl.ds(s * RPS, RPS)])

    @pl.when(s == NS - 1)
    def _flush_tail():
        pltpu.sync_copy(agg_sh.at[pl.ds(RPS * NS, RTAIL)],
                        out_hbm.at[c, pl.ds(RPS * NS, RTAIL)])


_sc_aggregate = pl.kernel(
    _sc_body,
    out_type=jax.ShapeDtypeStruct((NC, N, D), jnp.float32),
    mesh=plsc.VectorSubcoreMesh(
        core_axis_name="c", subcore_axis_name="s",
        num_cores=NC, num_subcores=NS),
    scratch_types=[
        pltpu.VMEM((2, K), jnp.int32),
        pltpu.VMEM((2, K), jnp.int32),
        pltpu.VMEM((EPW + 8,), jnp.float32),
        pltpu.VMEM((K, D), jnp.float32),
        pltpu.VMEM((K, D), jnp.float32),
        pltpu.VMEM_SHARED((N, D), jnp.float32),
        pltpu.SemaphoreType.DMA,
        pltpu.SemaphoreType.DMA,
    ],
    compiler_params=pltpu.CompilerParams(needs_layout_passes=False),
)


def _tc_head_body(agg_ref, batch_ref, wenc_ref, benc_ref, wcls_ref, bcls_ref,
                  out_ref):
    agg = agg_ref[0] + agg_ref[1]
    h = jnp.dot(agg, wenc_ref[...], preferred_element_type=jnp.float32)
    h = jnp.maximum(h + benc_ref[...], 0.0)
    bt = batch_ref[...]
    gids = lax.broadcasted_iota(jnp.int32, (G, N), 0)
    oh = (gids == bt).astype(jnp.float32)
    pooled_sum = jnp.dot(oh, h, preferred_element_type=jnp.float32)
    counts = jnp.sum(oh, axis=1, keepdims=True)
    pooled = pooled_sum / jnp.maximum(counts, 1.0)
    logits = jnp.dot(pooled, wcls_ref[...],
                     preferred_element_type=jnp.float32) + bcls_ref[...]
    out_ref[...] = logits


_tc_head = pl.pallas_call(
    _tc_head_body,
    out_shape=jax.ShapeDtypeStruct((G, C), jnp.float32),
)


def kernel(x, edge_index, batch, edge_weight, W_enc, b_enc, W_cls, b_cls):
    sd = jnp.stack([edge_index[0].reshape(NW, CHUNKS, K),
                    edge_index[1].reshape(NW, CHUNKS, K)], axis=2)
    zeros_nd = jnp.zeros((N, D), jnp.float32)
    agg2 = _sc_aggregate(x, sd, edge_weight, zeros_nd)
    return _tc_head(agg2, batch.reshape(1, N), W_enc, b_enc.reshape(1, D),
                    W_cls, b_cls.reshape(1, C))


# speedup vs baseline: 8.1563x; 1.6556x over previous
import functools

import jax
import jax.numpy as jnp
from jax import lax
from jax.experimental import pallas as pl
from jax.experimental.pallas import tpu as pltpu
from jax.experimental.pallas import tpu_sc as plsc

N = 10000
E = 320000
D = 128
C = 10
G = 64

NC = 2
NS = 16
NW = NC * NS
EPW = E // NW
K = 80
CHUNKS = EPW // K
RPS = 624
RTAIL = N - RPS * NS
LANES = 16


def _sc_body(x_hbm, sd_hbm, w_hbm, zero_hbm, out_hbm,
             sd0_v, sd1_v, w_v, rows0_v, rows1_v, agg_sh, sem0, sem1):
    c = lax.axis_index("c")
    s = lax.axis_index("s")
    wid = c * NS + s

    pltpu.sync_copy(w_hbm.at[pl.ds(wid * EPW, EPW)], w_v.at[pl.ds(8, EPW)])

    pltpu.sync_copy(zero_hbm.at[pl.ds(s * RPS, RPS)],
                    agg_sh.at[pl.ds(s * RPS, RPS)])

    @pl.when(s == NS - 1)
    def _zero_tail():
        pltpu.sync_copy(zero_hbm.at[pl.ds(RPS * NS, RTAIL)],
                        agg_sh.at[pl.ds(RPS * NS, RTAIL)])

    plsc.subcore_barrier()

    def load_sd(g, sd_v):
        pltpu.sync_copy(sd_hbm.at[wid, g], sd_v)

    def gather(sd_v, rows_v, sem):
        return pltpu.make_async_copy(x_hbm.at[sd_v.at[0]], rows_v, sem)

    def scale_and_scatter(g, rows_v, sd_v):
        wbase = g * K + 8

        @plsc.parallel_loop(0, K, unroll=4)
        def _scale(e):
            eidx = jnp.full((LANES,), 0, dtype=jnp.int32) + (e + wbase)
            wspl = plsc.load_gather(w_v, [eidx])
            for j in range(D // LANES):
                sl = rows_v[e, pl.ds(j * LANES, LANES)]
                rows_v[e, pl.ds(j * LANES, LANES)] = sl * wspl

        pltpu.sync_copy(rows_v, agg_sh.at[sd_v.at[1]], add=True)

    load_sd(0, sd0_v)
    gather(sd0_v, rows0_v, sem0).start()

    def pair_body(h, carry):
        g0 = h * 2
        load_sd(g0 + 1, sd1_v)
        gather(sd1_v, rows1_v, sem1).start()
        gather(sd0_v, rows0_v, sem0).wait()
        scale_and_scatter(g0, rows0_v, sd0_v)
        load_sd(g0 + 2, sd0_v)
        gather(sd0_v, rows0_v, sem0).start()
        gather(sd1_v, rows1_v, sem1).wait()
        scale_and_scatter(g0 + 1, rows1_v, sd1_v)
        return carry

    lax.fori_loop(0, CHUNKS // 2, pair_body, 0)
    gather(sd0_v, rows0_v, sem0).wait()
    scale_and_scatter(CHUNKS - 1, rows0_v, sd0_v)
    plsc.subcore_barrier()

    pltpu.sync_copy(agg_sh.at[pl.ds(s * RPS, RPS)],
                    out_hbm.at[c, pl.ds(s * RPS, RPS)])

    @pl.when(s == NS - 1)
    def _flush_tail():
        pltpu.sync_copy(agg_sh.at[pl.ds(RPS * NS, RTAIL)],
                        out_hbm.at[c, pl.ds(RPS * NS, RTAIL)])


_sc_aggregate = pl.kernel(
    _sc_body,
    out_type=jax.ShapeDtypeStruct((NC, N, D), jnp.float32),
    mesh=plsc.VectorSubcoreMesh(
        core_axis_name="c", subcore_axis_name="s",
        num_cores=NC, num_subcores=NS),
    scratch_types=[
        pltpu.VMEM((2, K), jnp.int32),
        pltpu.VMEM((2, K), jnp.int32),
        pltpu.VMEM((EPW + 8,), jnp.float32),
        pltpu.VMEM((K, D), jnp.float32),
        pltpu.VMEM((K, D), jnp.float32),
        pltpu.VMEM_SHARED((N, D), jnp.float32),
        pltpu.SemaphoreType.DMA,
        pltpu.SemaphoreType.DMA,
    ],
    compiler_params=pltpu.CompilerParams(needs_layout_passes=False),
)


def _tc_head_body(agg_ref, batch_ref, wenc_ref, benc_ref, wcls_ref, bcls_ref,
                  out_ref):
    agg = agg_ref[0] + agg_ref[1]
    h = jnp.dot(agg, wenc_ref[...], preferred_element_type=jnp.float32)
    h = jnp.maximum(h + benc_ref[...], 0.0)
    bt = batch_ref[...]
    gids = lax.broadcasted_iota(jnp.int32, (G, N), 0)
    oh = (gids == bt).astype(jnp.float32)
    pooled_sum = jnp.dot(oh, h, preferred_element_type=jnp.float32)
    counts = jnp.sum(oh, axis=1, keepdims=True)
    pooled = pooled_sum / jnp.maximum(counts, 1.0)
    logits = jnp.dot(pooled, wcls_ref[...],
                     preferred_element_type=jnp.float32) + bcls_ref[...]
    out_ref[...] = logits


_tc_head = pl.pallas_call(
    _tc_head_body,
    out_shape=jax.ShapeDtypeStruct((G, C), jnp.float32),
)


def kernel(x, edge_index, batch, edge_weight, W_enc, b_enc, W_cls, b_cls):
    sd = jnp.stack([edge_index[0].reshape(NW, CHUNKS, K),
                    edge_index[1].reshape(NW, CHUNKS, K)], axis=2)
    zeros_nd = jnp.zeros((N, D), jnp.float32)
    agg2 = _sc_aggregate(x, sd, edge_weight, zeros_nd)
    return _tc_head(agg2, batch.reshape(1, N), W_enc, b_enc.reshape(1, D),
                    W_cls, b_cls.reshape(1, C))
